# Initial kernel scaffold; baseline (speedup 1.0000x reference)
import jax
import jax.numpy as jnp
from jax.experimental import pallas as pl


def kernel(x, table):
    raise NotImplementedError("write your pallas kernel here")


# speedup vs baseline: 2.4891x; 2.4891x over previous
import jax
import jax.numpy as jnp
from jax import lax
from jax.experimental import pallas as pl
from jax.experimental.pallas import tpu as pltpu
from jax.experimental.pallas import tpu_sc as plsc

D = 16
B_TOTAL = 16384 * 200
NW = 32
B_PER_W = B_TOTAL // NW
C = 2048
N_CHUNKS = B_PER_W // C


def _embed_body(table_hbm, idx_hbm, out_hbm, idx_v, rows_v, sem):
    wid = lax.axis_index("s") * 2 + lax.axis_index("c")
    base = wid * B_PER_W

    def chunk(i, carry):
        off = base + i * C
        pltpu.sync_copy(idx_hbm.at[pl.ds(off, C)], idx_v)
        pltpu.async_copy(table_hbm.at[idx_v], rows_v, sem).wait()
        pltpu.sync_copy(rows_v, out_hbm.at[pl.ds(off, C)])
        return carry

    lax.fori_loop(0, N_CHUNKS, chunk, 0)


@jax.jit
def kernel(x, table):
    idx = x.reshape(-1).astype(jnp.int32)
    mesh = plsc.VectorSubcoreMesh(core_axis_name="c", subcore_axis_name="s")
    out = pl.kernel(
        _embed_body,
        out_type=jax.ShapeDtypeStruct((B_TOTAL, D), jnp.float32),
        mesh=mesh,
        scratch_types=[
            pltpu.VMEM((C,), jnp.int32),
            pltpu.VMEM((C, D), jnp.float32),
            pltpu.SemaphoreType.DMA,
        ],
        compiler_params=pltpu.CompilerParams(use_tc_tiling_on_sc=False),
    )(table, idx)
    return out.reshape(x.shape + (D,))


# speedup vs baseline: 2.5298x; 1.0164x over previous
import jax
import jax.numpy as jnp
from jax import lax
from jax.experimental import pallas as pl
from jax.experimental.pallas import tpu as pltpu
from jax.experimental.pallas import tpu_sc as plsc

D = 16
B_TOTAL = 16384 * 200
NW = 32
B_PER_W = B_TOTAL // NW
C = 2048
N_CHUNKS = B_PER_W // C


NBUF = 2
R = N_CHUNKS // NBUF


def _embed_body(table_hbm, idx_hbm, out_hbm,
                idx0, idx1, rows0, rows1,
                isem0, isem1, gsem0, gsem1, osem0, osem1):
    idxs = [idx0, idx1]
    rows = [rows0, rows1]
    isems = [isem0, isem1]
    gsems = [gsem0, gsem1]
    osems = [osem0, osem1]

    wid = lax.axis_index("s") * 2 + lax.axis_index("c")
    base = wid * B_PER_W

    def idx_start(i, b):
        pltpu.async_copy(idx_hbm.at[pl.ds(base + i * C, C)], idxs[b], isems[b])

    def idx_wait(i, b):
        pltpu.make_async_copy(
            idx_hbm.at[pl.ds(base + i * C, C)], idxs[b], isems[b]).wait()

    def gat_start(b):
        pltpu.async_copy(table_hbm.at[idxs[b]], rows[b], gsems[b])

    def gat_wait(b):
        pltpu.make_async_copy(table_hbm.at[idxs[b]], rows[b], gsems[b]).wait()

    def out_start(i, b):
        pltpu.async_copy(rows[b], out_hbm.at[pl.ds(base + i * C, C)], osems[b])

    def out_wait(i, b):
        pltpu.make_async_copy(
            rows[b], out_hbm.at[pl.ds(base + i * C, C)], osems[b]).wait()

    for b in range(NBUF):
        idx_start(b, b)
    for b in range(NBUF):
        idx_wait(b, b)
        gat_start(b)
        gat_wait(b)
        idx_start(b + NBUF, b)
        out_start(b, b)

    def round_body(g, carry):
        for b in range(NBUF):
            i = g * NBUF + b
            idx_wait(i, b)
            out_wait(i - NBUF, b)
            gat_start(b)
            gat_wait(b)
            idx_start(lax.rem(i + NBUF, N_CHUNKS), b)
            out_start(i, b)
        return carry

    lax.fori_loop(1, R, round_body, 0)

    for b in range(NBUF):
        i = (R - 1) * NBUF + b
        idx_wait(lax.rem(i + NBUF, N_CHUNKS), b)
        out_wait(i, b)


@jax.jit
def kernel(x, table):
    idx = x.reshape(-1).astype(jnp.int32)
    mesh = plsc.VectorSubcoreMesh(core_axis_name="c", subcore_axis_name="s")
    out = pl.kernel(
        _embed_body,
        out_type=jax.ShapeDtypeStruct((B_TOTAL, D), jnp.float32),
        mesh=mesh,
        scratch_types=[
            pltpu.VMEM((C,), jnp.int32),
            pltpu.VMEM((C,), jnp.int32),
            pltpu.VMEM((C, D), jnp.float32),
            pltpu.VMEM((C, D), jnp.float32),
            pltpu.SemaphoreType.DMA,
            pltpu.SemaphoreType.DMA,
            pltpu.SemaphoreType.DMA,
            pltpu.SemaphoreType.DMA,
            pltpu.SemaphoreType.DMA,
            pltpu.SemaphoreType.DMA,
        ],
        compiler_params=pltpu.CompilerParams(use_tc_tiling_on_sc=False),
    )(table, idx)
    return out.reshape(x.shape + (D,))


# speedup vs baseline: 2.5718x; 1.0166x over previous
import jax
import jax.numpy as jnp
from jax import lax
from jax.experimental import pallas as pl
from jax.experimental.pallas import tpu as pltpu
from jax.experimental.pallas import tpu_sc as plsc

D = 16
B_TOTAL = 16384 * 200
NW = 32
B_PER_W = B_TOTAL // NW
C = 512
N_CHUNKS = B_PER_W // C
NBUF = 8
L = 4
R = N_CHUNKS // NBUF


def _embed_body(table_hbm, idx_hbm, out_hbm, *bufs):
    idxs = bufs[0:NBUF]
    rows = bufs[NBUF:2 * NBUF]
    isems = bufs[2 * NBUF:3 * NBUF]
    gsems = bufs[3 * NBUF:4 * NBUF]
    osems = bufs[4 * NBUF:5 * NBUF]

    wid = lax.axis_index("s") * 2 + lax.axis_index("c")
    base = wid * B_PER_W

    def idx_start(i, b):
        pltpu.async_copy(idx_hbm.at[pl.ds(base + i * C, C)], idxs[b], isems[b])

    def idx_wait(i, b):
        pltpu.make_async_copy(
            idx_hbm.at[pl.ds(base + i * C, C)], idxs[b], isems[b]).wait()

    def gat_start(b):
        pltpu.async_copy(table_hbm.at[idxs[b]], rows[b], gsems[b])

    def gat_wait(b):
        pltpu.make_async_copy(table_hbm.at[idxs[b]], rows[b], gsems[b]).wait()

    def out_start(i, b):
        pltpu.async_copy(rows[b], out_hbm.at[pl.ds(base + i * C, C)], osems[b])

    def out_wait(i, b):
        pltpu.make_async_copy(
            rows[b], out_hbm.at[pl.ds(base + i * C, C)], osems[b]).wait()

    for b in range(NBUF):
        idx_start(b, b)
    for b in range(NBUF):
        i = b
        idx_wait(i, b)
        gat_start(b)
        if i >= L:
            j = i - L
            bj = j % NBUF
            gat_wait(bj)
            idx_start(j + NBUF, bj)
            out_start(j, bj)

    def round_body(g, carry):
        for b in range(NBUF):
            i = g * NBUF + b
            bj = (b + NBUF - L) % NBUF
            idx_wait(i, b)
            out_wait(i - NBUF, b)
            gat_start(b)
            gat_wait(bj)
            idx_start(lax.rem(i - L + NBUF, N_CHUNKS), bj)
            out_start(i - L, bj)
        return carry

    lax.fori_loop(1, R, round_body, 0)

    last = (R - 1) * NBUF
    for b in range(NBUF - L, NBUF):
        gat_wait(b)
        out_start(last + b, b)
    for b in range(L):
        idx_wait(b, b)
    for b in range(NBUF):
        out_wait(last + b, b)


@jax.jit
def kernel(x, table):
    idx = x.reshape(-1).astype(jnp.int32)
    mesh = plsc.VectorSubcoreMesh(core_axis_name="c", subcore_axis_name="s")
    out = pl.kernel(
        _embed_body,
        out_type=jax.ShapeDtypeStruct((B_TOTAL, D), jnp.float32),
        mesh=mesh,
        scratch_types=(
            [pltpu.VMEM((C,), jnp.int32) for _ in range(NBUF)]
            + [pltpu.VMEM((C, D), jnp.float32) for _ in range(NBUF)]
            + [pltpu.SemaphoreType.DMA for _ in range(3 * NBUF)]
        ),
        compiler_params=pltpu.CompilerParams(use_tc_tiling_on_sc=False),
    )(table, idx)
    return out.reshape(x.shape + (D,))


# speedup vs baseline: 3.2055x; 1.2464x over previous
import jax
import jax.numpy as jnp
from jax import lax
from jax.experimental import pallas as pl
from jax.experimental.pallas import tpu as pltpu
from jax.experimental.pallas import tpu_sc as plsc

D = 16
NB = 16384
NT = 200
NW = 32
BW = NB // NW
NBT = BW // 128
TSTRIDE = 2 * (NB // 128) * 8 * 128
FSTRIDE = (NB // 128) * 8 * 128


def _embed_body(table_hbm, idx_hbm, out_hbm, idx_v, rows_v, rt, sem):
    wid = lax.axis_index("s") * 2 + lax.axis_index("c")
    bbase = wid * BW
    iot = lax.iota(jnp.int32, 16)
    cvec = (iot // 8) * (NBT * 1024) + (iot % 8) * 128

    def per_t(t, carry):
        pltpu.sync_copy(idx_hbm.at[pl.ds(t * NB + bbase, BW)], idx_v)
        pltpu.async_copy(table_hbm.at[idx_v], rows_v, sem).wait()
        for r in range(BW):
            v = rows_v[r]
            plsc.store_scatter(rt, [cvec + ((r // 128) * 1024 + r % 128)], v)
        obase = t * TSTRIDE + wid * (NBT * 1024)
        pltpu.sync_copy(rt.at[pl.ds(0, NBT * 1024)],
                        out_hbm.at[pl.ds(obase, NBT * 1024)])
        pltpu.sync_copy(rt.at[pl.ds(NBT * 1024, NBT * 1024)],
                        out_hbm.at[pl.ds(obase + FSTRIDE, NBT * 1024)])
        return carry

    lax.fori_loop(0, NT, per_t, 0)


@jax.jit
def kernel(x, table):
    idx = x.T.reshape(-1).astype(jnp.int32)
    mesh = plsc.VectorSubcoreMesh(core_axis_name="c", subcore_axis_name="s")
    out = pl.kernel(
        _embed_body,
        out_type=jax.ShapeDtypeStruct((NT * 2 * (NB // 128) * 8 * 128,),
                                      jnp.float32),
        mesh=mesh,
        scratch_types=[
            pltpu.VMEM((BW,), jnp.int32),
            pltpu.VMEM((BW, D), jnp.float32),
            pltpu.VMEM((2 * NBT * 1024,), jnp.float32),
            pltpu.SemaphoreType.DMA,
        ],
        compiler_params=pltpu.CompilerParams(
            use_tc_tiling_on_sc=False, needs_layout_passes=False),
    )(table, idx)
    out = out.reshape(NT, 2, NB // 128, 8, 128)
    return out.transpose(2, 4, 0, 1, 3).reshape(NB, NT, D)


# speedup vs baseline: 3.9507x; 1.2324x over previous
import jax
import jax.numpy as jnp
from jax import lax
from jax.experimental import pallas as pl
from jax.experimental.pallas import tpu as pltpu
from jax.experimental.pallas import tpu_sc as plsc

D = 16
NB = 16384
NT = 200
NW = 32
BW = NB // NW
NBT = BW // 128
HT = NBT * 1024
TSTRIDE = 2 * (NB // 128) * 8 * 128
FSTRIDE = (NB // 128) * 8 * 128


def _embed_body(table_hbm, idx_hbm, out_hbm,
                idx0, idx1, rows0, rows1, rt0, rt1,
                g0, g1, o0, o1):
    wid = lax.axis_index("s") * 2 + lax.axis_index("c")
    bbase = wid * BW
    iot = lax.iota(jnp.int32, 16)
    cvec = (iot // 8) * HT + (iot % 8) * 128
    idx_b = [idx0, idx1]
    rows_b = [rows0, rows1]
    rt_b = [rt0, rt1]
    g_b = [g0, g1]
    o_b = [o0, o1]

    def out_copy(t, p, start):
        obase = t * TSTRIDE + wid * HT
        src0, src1 = rt_b[p].at[pl.ds(0, HT)], rt_b[p].at[pl.ds(HT, HT)]
        dst0 = out_hbm.at[pl.ds(obase, HT)]
        dst1 = out_hbm.at[pl.ds(obase + FSTRIDE, HT)]
        if start:
            pltpu.async_copy(src0, dst0, o_b[p])
            pltpu.async_copy(src1, dst1, o_b[p])
        else:
            pltpu.make_async_copy(src0, dst0, o_b[p]).wait()
            pltpu.make_async_copy(src1, dst1, o_b[p]).wait()

    pltpu.sync_copy(idx_hbm.at[pl.ds(bbase, BW)], idx0)
    pltpu.async_copy(table_hbm.at[idx0], rows0, g0)

    def step(t, p, q):
        @pl.when(t < NT - 1)
        def _prefetch():
            pltpu.sync_copy(idx_hbm.at[pl.ds((t + 1) * NB + bbase, BW)],
                            idx_b[q])
            pltpu.async_copy(table_hbm.at[idx_b[q]], rows_b[q], g_b[q])

        pltpu.make_async_copy(table_hbm.at[idx_b[p]], rows_b[p], g_b[p]).wait()

        @pl.when(t >= 2)
        def _drain():
            out_copy(t - 2, p, start=False)

        def per_grp(g, c):
            idxbase = cvec + ((g // 8) * 1024 + (g % 8) * 16)
            rbase = g * 16
            for j in range(16):
                v = rows_b[p][rbase + j]
                plsc.store_scatter(rt_b[p], [idxbase + j], v)
            return c

        lax.fori_loop(0, BW // 16, per_grp, 0)
        out_copy(t, p, start=True)

    def per_i(i, carry):
        step(2 * i, 0, 1)
        step(2 * i + 1, 1, 0)
        return carry

    lax.fori_loop(0, NT // 2, per_i, 0)

    out_copy(NT - 2, (NT - 2) % 2, start=False)
    out_copy(NT - 1, (NT - 1) % 2, start=False)


@jax.jit
def kernel(x, table):
    idx = x.T.reshape(-1).astype(jnp.int32)
    mesh = plsc.VectorSubcoreMesh(core_axis_name="c", subcore_axis_name="s")
    out = pl.kernel(
        _embed_body,
        out_type=jax.ShapeDtypeStruct((NT * TSTRIDE,), jnp.float32),
        mesh=mesh,
        scratch_types=[
            pltpu.VMEM((BW,), jnp.int32),
            pltpu.VMEM((BW,), jnp.int32),
            pltpu.VMEM((BW, D), jnp.float32),
            pltpu.VMEM((BW, D), jnp.float32),
            pltpu.VMEM((2 * HT,), jnp.float32),
            pltpu.VMEM((2 * HT,), jnp.float32),
            pltpu.SemaphoreType.DMA,
            pltpu.SemaphoreType.DMA,
            pltpu.SemaphoreType.DMA,
            pltpu.SemaphoreType.DMA,
        ],
        compiler_params=pltpu.CompilerParams(
            use_tc_tiling_on_sc=False, needs_layout_passes=False),
    )(table, idx)
    out = out.reshape(NT, 2, NB // 128, 8, 128)
    return out.transpose(2, 4, 0, 1, 3).reshape(NB, NT, D)


# speedup vs baseline: 4.6322x; 1.1725x over previous
import jax
import jax.numpy as jnp
from jax import lax
from jax.experimental import pallas as pl
from jax.experimental.pallas import tpu as pltpu
from jax.experimental.pallas import tpu_sc as plsc

D = 16
NB = 16384
NT = 200
NW = 32
BW = NB // NW
NBT = BW // 128
HT = NBT * 1024
TSTRIDE = 2 * (NB // 128) * 8 * 128
FSTRIDE = (NB // 128) * 8 * 128


def _embed_body(table_hbm, idx_hbm, out_hbm,
                idx0, idx1, rows0, rows1, rt0, rt1,
                g0, g1, o0, o1):
    wid = lax.axis_index("s") * 2 + lax.axis_index("c")
    bbase = wid * BW
    iot = lax.iota(jnp.int32, 16)
    cvec = (iot // 8) * HT + (iot % 8) * 128
    idx_b = [idx0, idx1]
    rows_b = [rows0, rows1]
    rt_b = [rt0, rt1]
    g_b = [g0, g1]
    o_b = [o0, o1]

    def out_copy(t, p, start):
        obase = t * TSTRIDE + wid * HT
        src0, src1 = rt_b[p].at[pl.ds(0, HT)], rt_b[p].at[pl.ds(HT, HT)]
        dst0 = out_hbm.at[pl.ds(obase, HT)]
        dst1 = out_hbm.at[pl.ds(obase + FSTRIDE, HT)]
        if start:
            pltpu.async_copy(src0, dst0, o_b[p])
            pltpu.async_copy(src1, dst1, o_b[p])
        else:
            pltpu.make_async_copy(src0, dst0, o_b[p]).wait()
            pltpu.make_async_copy(src1, dst1, o_b[p]).wait()

    pltpu.sync_copy(idx_hbm.at[pl.ds(bbase, BW)], idx0)
    pltpu.async_copy(table_hbm.at[idx0], rows0, g0)

    def step(t, p, q):
        @pl.when(t < NT - 1)
        def _prefetch():
            pltpu.sync_copy(idx_hbm.at[pl.ds((t + 1) * NB + bbase, BW)],
                            idx_b[q])
            pltpu.async_copy(table_hbm.at[idx_b[q]], rows_b[q], g_b[q])

        pltpu.make_async_copy(table_hbm.at[idx_b[p]], rows_b[p], g_b[p]).wait()

        @pl.when(t >= 2)
        def _drain():
            out_copy(t - 2, p, start=False)

        def per_grp(g, c):
            idxbase = cvec + ((g // 8) * 1024 + (g % 8) * 16)
            rbase = g * 16
            vs = [rows_b[p][rbase + j] for j in range(16)]
            for j in range(16):
                plsc.store_scatter(rt_b[p], [idxbase + j], vs[j])
            return c

        lax.fori_loop(0, BW // 16, per_grp, 0)
        out_copy(t, p, start=True)

    def per_i(i, carry):
        step(2 * i, 0, 1)
        step(2 * i + 1, 1, 0)
        return carry

    lax.fori_loop(0, NT // 2, per_i, 0)

    out_copy(NT - 2, (NT - 2) % 2, start=False)
    out_copy(NT - 1, (NT - 1) % 2, start=False)


@jax.jit
def kernel(x, table):
    idx = x.T.reshape(-1).astype(jnp.int32)
    mesh = plsc.VectorSubcoreMesh(core_axis_name="c", subcore_axis_name="s")
    out = pl.kernel(
        _embed_body,
        out_type=jax.ShapeDtypeStruct((NT * TSTRIDE,), jnp.float32),
        mesh=mesh,
        scratch_types=[
            pltpu.VMEM((BW,), jnp.int32),
            pltpu.VMEM((BW,), jnp.int32),
            pltpu.VMEM((BW, D), jnp.float32),
            pltpu.VMEM((BW, D), jnp.float32),
            pltpu.VMEM((2 * HT,), jnp.float32),
            pltpu.VMEM((2 * HT,), jnp.float32),
            pltpu.SemaphoreType.DMA,
            pltpu.SemaphoreType.DMA,
            pltpu.SemaphoreType.DMA,
            pltpu.SemaphoreType.DMA,
        ],
        compiler_params=pltpu.CompilerParams(
            use_tc_tiling_on_sc=False, needs_layout_passes=False),
    )(table, idx)
    out = out.reshape(NT, 2, NB // 128, 8, 128)
    return out.transpose(2, 4, 0, 1, 3).reshape(NB, NT, D)


# speedup vs baseline: 4.8292x; 1.0425x over previous
import jax
import jax.numpy as jnp
from jax import lax
from jax.experimental import pallas as pl
from jax.experimental.pallas import tpu as pltpu
from jax.experimental.pallas import tpu_sc as plsc

D = 16
NB = 16384
NT = 200
NW = 32
BW = NB // NW
NBT = BW // 128
HT = NBT * 1024
TSTRIDE = 2 * (NB // 128) * 8 * 128
FSTRIDE = (NB // 128) * 8 * 128


def _embed_body(table_hbm, idx_hbm, out_hbm,
                idx0, idx1, rows0, rows1, rt0, rt1,
                g0, g1, o0, o1):
    wid = lax.axis_index("s") * 2 + lax.axis_index("c")
    bbase = wid * BW
    iot = lax.iota(jnp.int32, 16)
    cvec = (iot // 8) * HT + (iot % 8) * 128
    idx_b = [idx0, idx1]
    rows_b = [rows0, rows1]
    rt_b = [rt0, rt1]
    g_b = [g0, g1]
    o_b = [o0, o1]

    def out_copy(t, p, start):
        obase = t * TSTRIDE + wid * HT
        src0, src1 = rt_b[p].at[pl.ds(0, HT)], rt_b[p].at[pl.ds(HT, HT)]
        dst0 = out_hbm.at[pl.ds(obase, HT)]
        dst1 = out_hbm.at[pl.ds(obase + FSTRIDE, HT)]
        if start:
            pltpu.async_copy(src0, dst0, o_b[p])
            pltpu.async_copy(src1, dst1, o_b[p])
        else:
            pltpu.make_async_copy(src0, dst0, o_b[p]).wait()
            pltpu.make_async_copy(src1, dst1, o_b[p]).wait()

    pltpu.sync_copy(idx_hbm.at[pl.ds(bbase, BW)], idx0)
    pltpu.async_copy(table_hbm.at[idx0], rows0, g0)

    def step(t, p, q):
        @pl.when(t < NT - 1)
        def _prefetch():
            pltpu.sync_copy(idx_hbm.at[pl.ds((t + 1) * NB + bbase, BW)],
                            idx_b[q])
            pltpu.async_copy(table_hbm.at[idx_b[q]], rows_b[q], g_b[q])

        pltpu.make_async_copy(table_hbm.at[idx_b[p]], rows_b[p], g_b[p]).wait()

        @pl.when(t >= 2)
        def _drain():
            out_copy(t - 2, p, start=False)

        @plsc.parallel_loop(0, BW // 16, unroll=2)
        def _transpose(g):
            idxbase = cvec + ((g // 8) * 1024 + (g % 8) * 16)
            rbase = g * 16
            vs = [rows_b[p][rbase + j] for j in range(16)]
            for j in range(16):
                plsc.store_scatter(rt_b[p], [idxbase + j], vs[j])

        out_copy(t, p, start=True)

    def per_i(i, carry):
        step(2 * i, 0, 1)
        step(2 * i + 1, 1, 0)
        return carry

    lax.fori_loop(0, NT // 2, per_i, 0)

    out_copy(NT - 2, (NT - 2) % 2, start=False)
    out_copy(NT - 1, (NT - 1) % 2, start=False)


@jax.jit
def kernel(x, table):
    idx = x.T.reshape(-1).astype(jnp.int32)
    mesh = plsc.VectorSubcoreMesh(core_axis_name="c", subcore_axis_name="s")
    out = pl.kernel(
        _embed_body,
        out_type=jax.ShapeDtypeStruct((NT * TSTRIDE,), jnp.float32),
        mesh=mesh,
        scratch_types=[
            pltpu.VMEM((BW,), jnp.int32),
            pltpu.VMEM((BW,), jnp.int32),
            pltpu.VMEM((BW, D), jnp.float32),
            pltpu.VMEM((BW, D), jnp.float32),
            pltpu.VMEM((2 * HT,), jnp.float32),
            pltpu.VMEM((2 * HT,), jnp.float32),
            pltpu.SemaphoreType.DMA,
            pltpu.SemaphoreType.DMA,
            pltpu.SemaphoreType.DMA,
            pltpu.SemaphoreType.DMA,
        ],
        compiler_params=pltpu.CompilerParams(
            use_tc_tiling_on_sc=False, needs_layout_passes=False),
    )(table, idx)
    out = out.reshape(NT, 2, NB // 128, 8, 128)
    return out.transpose(2, 4, 0, 1, 3).reshape(NB, NT, D)


# speedup vs baseline: 5.3149x; 1.1006x over previous
import jax
import jax.numpy as jnp
from jax import lax
from jax.experimental import pallas as pl
from jax.experimental.pallas import tpu as pltpu
from jax.experimental.pallas import tpu_sc as plsc

D = 16
NB = 16384
NT = 200
NW = 32
BW = NB // NW
NBT = BW // 128
HT = NBT * 1024
TSTRIDE = 2 * (NB // 128) * 8 * 128
FSTRIDE = (NB // 128) * 8 * 128


def _embed_body(table_hbm, idx_hbm, out_hbm,
                idx0, idx1, rows0, rows1, rt0, rt1,
                g0, g1, o0, o1, i0, i1):
    wid = lax.axis_index("s") * 2 + lax.axis_index("c")
    bbase = wid * BW
    iot = lax.iota(jnp.int32, 16)
    cvec = (iot // 8) * HT + (iot % 8) * 128
    idx_b = [idx0, idx1]
    rows_b = [rows0, rows1]
    rt_b = [rt0, rt1]
    g_b = [g0, g1]
    o_b = [o0, o1]
    i_b = [i0, i1]

    def idx_copy(t, p, start):
        src = idx_hbm.at[pl.ds(t * NB + bbase, BW)]
        if start:
            pltpu.async_copy(src, idx_b[p], i_b[p])
        else:
            pltpu.make_async_copy(src, idx_b[p], i_b[p]).wait()

    def out_copy(t, p, start):
        obase = t * TSTRIDE + wid * HT
        src0, src1 = rt_b[p].at[pl.ds(0, HT)], rt_b[p].at[pl.ds(HT, HT)]
        dst0 = out_hbm.at[pl.ds(obase, HT)]
        dst1 = out_hbm.at[pl.ds(obase + FSTRIDE, HT)]
        if start:
            pltpu.async_copy(src0, dst0, o_b[p])
            pltpu.async_copy(src1, dst1, o_b[p])
        else:
            pltpu.make_async_copy(src0, dst0, o_b[p]).wait()
            pltpu.make_async_copy(src1, dst1, o_b[p]).wait()

    pltpu.sync_copy(idx_hbm.at[pl.ds(bbase, BW)], idx0)
    pltpu.async_copy(table_hbm.at[idx0], rows0, g0)
    idx_copy(1, 1, start=True)

    def step(t, p, q):
        @pl.when(t < NT - 1)
        def _prefetch():
            idx_copy(t + 1, q, start=False)
            pltpu.async_copy(table_hbm.at[idx_b[q]], rows_b[q], g_b[q])

        pltpu.make_async_copy(table_hbm.at[idx_b[p]], rows_b[p], g_b[p]).wait()

        @pl.when(t < NT - 2)
        def _iprefetch():
            idx_copy(t + 2, p, start=True)

        @pl.when(t >= 2)
        def _drain():
            out_copy(t - 2, p, start=False)

        @plsc.parallel_loop(0, BW // 16, unroll=2)
        def _transpose(g):
            idxbase = cvec + ((g // 8) * 1024 + (g % 8) * 16)
            rbase = g * 16
            vs = [rows_b[p][rbase + j] for j in range(16)]
            for j in range(16):
                plsc.store_scatter(rt_b[p], [idxbase + j], vs[j])

        out_copy(t, p, start=True)

    def per_i(i, carry):
        step(2 * i, 0, 1)
        step(2 * i + 1, 1, 0)
        return carry

    lax.fori_loop(0, NT // 2, per_i, 0)

    out_copy(NT - 2, (NT - 2) % 2, start=False)
    out_copy(NT - 1, (NT - 1) % 2, start=False)


@jax.jit
def kernel(x, table):
    idx = x.T.reshape(-1).astype(jnp.int32)
    mesh = plsc.VectorSubcoreMesh(core_axis_name="c", subcore_axis_name="s")
    out = pl.kernel(
        _embed_body,
        out_type=jax.ShapeDtypeStruct((NT * TSTRIDE,), jnp.float32),
        mesh=mesh,
        scratch_types=[
            pltpu.VMEM((BW,), jnp.int32),
            pltpu.VMEM((BW,), jnp.int32),
            pltpu.VMEM((BW, D), jnp.float32),
            pltpu.VMEM((BW, D), jnp.float32),
            pltpu.VMEM((2 * HT,), jnp.float32),
            pltpu.VMEM((2 * HT,), jnp.float32),
            pltpu.SemaphoreType.DMA,
            pltpu.SemaphoreType.DMA,
            pltpu.SemaphoreType.DMA,
            pltpu.SemaphoreType.DMA,
            pltpu.SemaphoreType.DMA,
            pltpu.SemaphoreType.DMA,
        ],
        compiler_params=pltpu.CompilerParams(
            use_tc_tiling_on_sc=False, needs_layout_passes=False),
    )(table, idx)
    out = out.reshape(NT, 2, NB // 128, 8, 128)
    return out.transpose(2, 4, 0, 1, 3).reshape(NB, NT, D)
